# Initial kernel scaffold; baseline (speedup 1.0000x reference)
import jax
import jax.numpy as jnp
from jax.experimental import pallas as pl


def kernel(x, edge_index, original, y, nodes, variants, W1, b1, W2, b2, Wd1, bd1, Wd2, bd2):
    raise NotImplementedError("write your pallas kernel here")


# speedup vs baseline: 3.4557x; 3.4557x over previous
import functools

import jax
import jax.numpy as jnp
from jax import lax
from jax.experimental import pallas as pl
from jax.experimental.pallas import tpu as pltpu
from jax.experimental.pallas import tpu_sc as plsc

N = 10000
E = 320000
P = 100000
F = 128
OUT = 2
FE = F + 16
NC = 2
NS = 16
NW = NC * NS
L = 16

NP = 10240
ROWS_PER_TILE = NP // NS
EDGE_CHUNK = 80
EDGES_PER_W = E // NW
EDGE_CHUNKS = EDGES_PER_W // EDGE_CHUNK

VPW = 3136
PP = VPW * NW
VCHUNK = 64
VCHUNKS = VPW // VCHUNK
D3 = 3 * F

_mesh = plsc.VectorSubcoreMesh(core_axis_name="c", subcore_axis_name="s",
                               num_cores=NC, num_subcores=NS)


def _wid():
    return lax.axis_index("s") * NC + lax.axis_index("c")


def _make_edge_agg():
    @functools.partial(
        pl.kernel,
        mesh=_mesh,
        compiler_params=pltpu.CompilerParams(needs_layout_passes=False),
        out_type=jax.ShapeDtypeStruct((NC, NP, F), jnp.float32),
        scratch_types=[
            pltpu.VMEM((EDGE_CHUNK,), jnp.int32),
            pltpu.VMEM((EDGE_CHUNK,), jnp.int32),
            pltpu.VMEM((EDGE_CHUNK, F), jnp.float32),
            pltpu.VMEM_SHARED((NP, F), jnp.float32),
            pltpu.SemaphoreType.DMA,
        ],
    )
    def edge_agg(table_hbm, src_hbm, dst_hbm, out_hbm,
                 sidx_v, didx_v, rows_v, acc_sh, sem):
        cid = lax.axis_index("c")
        tid = lax.axis_index("s")
        wid = tid * NC + cid
        zbase = pl.multiple_of(tid * ROWS_PER_TILE, 8)
        zv = jnp.zeros((L,), jnp.float32)

        def zrow(r, carry):
            for q in range(F // L):
                rows_v[r, pl.ds(q * L, L)] = zv
            return carry

        lax.fori_loop(0, EDGE_CHUNK, zrow, 0)
        for q in range(ROWS_PER_TILE // EDGE_CHUNK):
            pltpu.sync_copy(
                rows_v, acc_sh.at[pl.ds(zbase + q * EDGE_CHUNK, EDGE_CHUNK)])
        plsc.subcore_barrier()

        def chunk_body(j, carry):
            base = pl.multiple_of(wid * EDGES_PER_W + j * EDGE_CHUNK, 8)
            pltpu.sync_copy(src_hbm.at[pl.ds(base, EDGE_CHUNK)], sidx_v)
            pltpu.sync_copy(dst_hbm.at[pl.ds(base, EDGE_CHUNK)], didx_v)
            pltpu.async_copy(table_hbm.at[sidx_v], rows_v, sem).wait()
            pltpu.sync_copy(rows_v, acc_sh.at[didx_v], add=True)
            return carry

        lax.fori_loop(0, EDGE_CHUNKS, chunk_body, 0)
        plsc.subcore_barrier()
        for q in range(ROWS_PER_TILE // EDGE_CHUNK):
            rb = zbase + q * EDGE_CHUNK
            pltpu.sync_copy(acc_sh.at[pl.ds(rb, EDGE_CHUNK)], rows_v)
            pltpu.sync_copy(rows_v, out_hbm.at[cid, pl.ds(rb, EDGE_CHUNK)])

    return edge_agg


_edge_agg_f = _make_edge_agg()

NH = NP // F
_EROWS = 625
_EBLK = 25


def _deg_body(d_ref, out_ref):
    i = pl.program_id(0)
    d = d_ref[...].reshape(-1)
    hi = d // F
    lo = d - hi * F
    a = (hi[:, None] == lax.broadcasted_iota(jnp.int32, (1, NH), 1)
         ).astype(jnp.float32)
    b = (lo[:, None] == lax.broadcasted_iota(jnp.int32, (1, F), 1)
         ).astype(jnp.float32)
    blk = lax.dot_general(a, b, (((0,), (0,)), ((), ())),
                          preferred_element_type=jnp.float32)

    @pl.when(i == 0)
    def _():
        out_ref[...] = jnp.zeros_like(out_ref)

    out_ref[...] += blk


def _deg_mm(dst):
    return pl.pallas_call(
        _deg_body,
        grid=(_EROWS // _EBLK,),
        in_specs=[pl.BlockSpec((1, _EBLK, 512), lambda i: (i, 0, 0))],
        out_specs=pl.BlockSpec((NH, F), lambda i: (0, 0)),
        out_shape=jax.ShapeDtypeStruct((NH, F), jnp.float32),
    )(dst.reshape(_EROWS // _EBLK, _EBLK, 512))


_BN = 1000


def _layer1_body(x_ref, aggp_ref, deg_ref, w_ref, b_ref, h_ref, invd_ref):
    s = aggp_ref[0] + aggp_ref[1]
    invd = 1.0 / jnp.maximum(deg_ref[...], 1.0)
    agg = s * invd
    h = jnp.maximum(
        jnp.dot(x_ref[...] + agg, w_ref[...],
                preferred_element_type=jnp.float32) + b_ref[...], 0.0)
    h_ref[...] = h
    invd_ref[...] = jnp.broadcast_to(invd, (invd.shape[0], 8))


def _layer1(x, aggp, degp, W1, b1):
    return pl.pallas_call(
        _layer1_body,
        grid=(N // _BN,),
        in_specs=[
            pl.BlockSpec((_BN, F), lambda i: (i, 0)),
            pl.BlockSpec((NC, _BN, F), lambda i: (0, i, 0)),
            pl.BlockSpec((_BN, 1), lambda i: (i, 0)),
            pl.BlockSpec((F, F), lambda i: (0, 0)),
            pl.BlockSpec((1, F), lambda i: (0, 0)),
        ],
        out_specs=[
            pl.BlockSpec((_BN, F), lambda i: (i, 0)),
            pl.BlockSpec((_BN, 8), lambda i: (i, 0)),
        ],
        out_shape=[
            jax.ShapeDtypeStruct((N, F), jnp.float32),
            jax.ShapeDtypeStruct((N, 8), jnp.float32),
        ],
    )(x, aggp, degp, W1, b1.reshape(1, F))


def _layer2_body(h_ref, aggp_ref, invd_ref, w_ref, b_ref,
                 wa_ref, wb_ref, wc_ref, bd1_ref, a_ref, b2_ref, c_ref):
    s = aggp_ref[0] + aggp_ref[1]
    agg = s * invd_ref[:, 0:1]
    h2 = jnp.maximum(
        jnp.dot(h_ref[...] + agg, w_ref[...],
                preferred_element_type=jnp.float32) + b_ref[...], 0.0)
    a_ref[...] = jnp.dot(h2, wa_ref[...],
                         preferred_element_type=jnp.float32) + bd1_ref[...]
    b2_ref[...] = jnp.dot(h2, wb_ref[...], preferred_element_type=jnp.float32)
    c_ref[...] = jnp.dot(h2, wc_ref[...], preferred_element_type=jnp.float32)


def _layer2(h, aggp, invd, W2, b2, Wd1, bd1):
    return pl.pallas_call(
        _layer2_body,
        grid=(N // _BN,),
        in_specs=[
            pl.BlockSpec((_BN, F), lambda i: (i, 0)),
            pl.BlockSpec((NC, _BN, F), lambda i: (0, i, 0)),
            pl.BlockSpec((_BN, 8), lambda i: (i, 0)),
            pl.BlockSpec((F, F), lambda i: (0, 0)),
            pl.BlockSpec((1, F), lambda i: (0, 0)),
            pl.BlockSpec((F, D3), lambda i: (0, 0)),
            pl.BlockSpec((F, D3), lambda i: (0, 0)),
            pl.BlockSpec((F, D3), lambda i: (0, 0)),
            pl.BlockSpec((1, D3), lambda i: (0, 0)),
        ],
        out_specs=[
            pl.BlockSpec((_BN, D3), lambda i: (i, 0)),
            pl.BlockSpec((_BN, D3), lambda i: (i, 0)),
            pl.BlockSpec((_BN, D3), lambda i: (i, 0)),
        ],
        out_shape=[
            jax.ShapeDtypeStruct((N, D3), jnp.float32),
            jax.ShapeDtypeStruct((N, D3), jnp.float32),
            jax.ShapeDtypeStruct((N, D3), jnp.float32),
        ],
    )(h, aggp, invd, W2, b2.reshape(1, F),
      Wd1[:F], Wd1[F:2 * F], Wd1[2 * F:], bd1.reshape(1, D3))


@functools.partial(
    pl.kernel,
    mesh=_mesh,
    compiler_params=pltpu.CompilerParams(needs_layout_passes=False),
    out_type=jax.ShapeDtypeStruct((PP, D3), jnp.float32),
    scratch_types=[
        pltpu.VMEM((VCHUNK,), jnp.int32),
        pltpu.VMEM((VCHUNK,), jnp.int32),
        pltpu.VMEM((VCHUNK,), jnp.int32),
        pltpu.VMEM((VCHUNK, D3), jnp.float32),
        pltpu.VMEM((VCHUNK, D3), jnp.float32),
        pltpu.VMEM((VCHUNK, D3), jnp.float32),
        pltpu.SemaphoreType.DMA,
    ],
)
def _gather_sum(a_hbm, b_hbm, c_hbm, pi_hbm, si_hbm, di_hbm, out_hbm,
                pi_v, si_v, di_v, a_v, b_v, c_v, sem):
    cid = lax.axis_index("c")
    tid = lax.axis_index("s")
    wid = tid * NC + cid

    def chunk_body(ch, carry):
        base = pl.multiple_of(wid * VPW + ch * VCHUNK, 8)
        pltpu.sync_copy(pi_hbm.at[pl.ds(base, VCHUNK)], pi_v)
        pltpu.sync_copy(si_hbm.at[pl.ds(base, VCHUNK)], si_v)
        pltpu.sync_copy(di_hbm.at[pl.ds(base, VCHUNK)], di_v)
        pltpu.async_copy(a_hbm.at[pi_v], a_v, sem).wait()
        pltpu.async_copy(b_hbm.at[si_v], b_v, sem).wait()
        pltpu.async_copy(c_hbm.at[di_v], c_v, sem).wait()

        def sum_row(r, c2):
            for q in range(D3 // L):
                sl = pl.ds(q * L, L)
                a_v[r, sl] = a_v[r, sl] + b_v[r, sl] + c_v[r, sl]
            return c2

        lax.fori_loop(0, VCHUNK, sum_row, 0)
        pltpu.sync_copy(a_v, out_hbm.at[pl.ds(base, VCHUNK)])
        return carry

    lax.fori_loop(0, VCHUNKS, chunk_body, 0)


_VB = 3136


def _decode_body(hs_ref, pi_ref, wd2_ref, bd2_ref, m0_ref, m1_ref):
    i = pl.program_id(0)
    hid = jnp.maximum(hs_ref[...], 0.0)
    logits = jnp.dot(hid, wd2_ref[...],
                     preferred_element_type=jnp.float32) + bd2_ref[...]
    lm = jnp.max(logits, axis=1, keepdims=True)
    lse = lm + jnp.log(jnp.sum(jnp.exp(logits - lm), axis=1, keepdims=True))
    logp = logits - lse
    gidx = i * _VB + lax.broadcasted_iota(jnp.int32, (_VB, 1), 0)
    logp = jnp.where(gidx < P, logp, 0.0)
    pv = pi_ref[...].reshape(-1)
    hi = pv // F
    lo = pv - hi * F
    a = (hi[:, None] == lax.broadcasted_iota(jnp.int32, (1, NH), 1)
         ).astype(jnp.float32)
    b = (lo[:, None] == lax.broadcasted_iota(jnp.int32, (1, F), 1)
         ).astype(jnp.float32)
    m0 = lax.dot_general(a, b * logp[:, 0:1], (((0,), (0,)), ((), ())),
                         preferred_element_type=jnp.float32)
    m1 = lax.dot_general(a, b * logp[:, 1:2], (((0,), (0,)), ((), ())),
                         preferred_element_type=jnp.float32)

    @pl.when(i == 0)
    def _():
        m0_ref[...] = jnp.zeros_like(m0_ref)
        m1_ref[...] = jnp.zeros_like(m1_ref)

    m0_ref[...] += m0
    m1_ref[...] += m1


def _decode(hidsum, pi, Wd2, bd2):
    return pl.pallas_call(
        _decode_body,
        grid=(PP // _VB,),
        in_specs=[
            pl.BlockSpec((_VB, D3), lambda i: (i, 0)),
            pl.BlockSpec((1, 1, _VB), lambda i: (i, 0, 0)),
            pl.BlockSpec((D3, OUT), lambda i: (0, 0)),
            pl.BlockSpec((1, OUT), lambda i: (0, 0)),
        ],
        out_specs=[
            pl.BlockSpec((NH, F), lambda i: (0, 0)),
            pl.BlockSpec((NH, F), lambda i: (0, 0)),
        ],
        out_shape=[
            jax.ShapeDtypeStruct((NH, F), jnp.float32),
            jax.ShapeDtypeStruct((NH, F), jnp.float32),
        ],
    )(hidsum, pi.reshape(PP // _VB, 1, _VB), Wd2, bd2.reshape(1, OUT))


def _first(x):
    return x[0] if isinstance(x, (tuple, list)) else x


def kernel(x, edge_index, original, y, nodes, variants,
           W1, b1, W2, b2, Wd1, bd1, Wd2, bd2):
    del original, y, nodes
    src = edge_index[0]
    dst = edge_index[1]

    aggp1 = _first(_edge_agg_f(x, src, dst))
    deg = _deg_mm(dst).reshape(NP, 1)
    h, invd = _layer1(x, aggp1, deg, W1, b1)
    aggp2 = _first(_edge_agg_f(h, src, dst))
    A, B, C = _layer2(h, aggp2, invd, W2, b2, Wd1, bd1)

    pad = jnp.zeros((PP - P,), jnp.int32)
    pi = jnp.concatenate([variants[0], pad])
    si = jnp.concatenate([variants[1], pad])
    di = jnp.concatenate([variants[2], pad])
    hidsum = _first(_gather_sum(A, B, C, pi, si, di))
    m0, m1 = _decode(hidsum, pi, Wd2, bd2)
    return jnp.stack([m0.reshape(NP)[:N], m1.reshape(NP)[:N]], axis=1)


# speedup vs baseline: 5.2107x; 1.5079x over previous
import functools

import jax
import jax.numpy as jnp
from jax import lax
from jax.experimental import pallas as pl
from jax.experimental.pallas import tpu as pltpu
from jax.experimental.pallas import tpu_sc as plsc

N = 10000
E = 320000
P = 100000
F = 128
OUT = 2
FE = F + 16
NC = 2
NS = 16
NW = NC * NS
L = 16

NP = 10240
ROWS_PER_TILE = NP // NS
EDGE_CHUNK = 80
EDGES_PER_W = E // NW
EDGE_CHUNKS = EDGES_PER_W // EDGE_CHUNK

VPW = 3136
PP = VPW * NW
VCHUNK = 32
VCHUNKS = VPW // VCHUNK
D3 = 3 * F

_mesh = plsc.VectorSubcoreMesh(core_axis_name="c", subcore_axis_name="s",
                               num_cores=NC, num_subcores=NS)


def _wid():
    return lax.axis_index("s") * NC + lax.axis_index("c")


def _make_edge_agg():
    @functools.partial(
        pl.kernel,
        mesh=_mesh,
        compiler_params=pltpu.CompilerParams(needs_layout_passes=False),
        out_type=jax.ShapeDtypeStruct((NC, NP, F), jnp.float32),
        scratch_types=[
            pltpu.VMEM((EDGE_CHUNK,), jnp.int32),
            pltpu.VMEM((EDGE_CHUNK,), jnp.int32),
            pltpu.VMEM((EDGE_CHUNK,), jnp.int32),
            pltpu.VMEM((EDGE_CHUNK,), jnp.int32),
            pltpu.VMEM((EDGE_CHUNK, F), jnp.float32),
            pltpu.VMEM((EDGE_CHUNK, F), jnp.float32),
            pltpu.VMEM_SHARED((NP, F), jnp.float32),
            pltpu.SemaphoreType.DMA,
            pltpu.SemaphoreType.DMA,
        ],
    )
    def edge_agg(table_hbm, src_hbm, dst_hbm, out_hbm,
                 sidx0, didx0, sidx1, didx1, rows0, rows1, acc_sh,
                 sem0, sem1):
        cid = lax.axis_index("c")
        tid = lax.axis_index("s")
        wid = tid * NC + cid
        wbase = wid * EDGES_PER_W
        zbase = pl.multiple_of(tid * ROWS_PER_TILE, 8)
        zv = jnp.zeros((L,), jnp.float32)

        def zrow(r, carry):
            for q in range(F // L):
                rows0[r, pl.ds(q * L, L)] = zv
            return carry

        lax.fori_loop(0, EDGE_CHUNK, zrow, 0)
        for q in range(ROWS_PER_TILE // EDGE_CHUNK):
            pltpu.sync_copy(
                rows0, acc_sh.at[pl.ds(zbase + q * EDGE_CHUNK, EDGE_CHUNK)])
        plsc.subcore_barrier()

        def fetch(ch, sidx, didx, rows, sem):
            base = pl.multiple_of(wbase + ch * EDGE_CHUNK, 8)
            pltpu.sync_copy(src_hbm.at[pl.ds(base, EDGE_CHUNK)], sidx)
            pltpu.sync_copy(dst_hbm.at[pl.ds(base, EDGE_CHUNK)], didx)
            return pltpu.async_copy(table_hbm.at[sidx], rows, sem)

        fetch(0, sidx0, didx0, rows0, sem0)

        def pair_body(j2, carry):
            ch = 2 * j2
            fetch(ch + 1, sidx1, didx1, rows1, sem1)
            pltpu.make_async_copy(table_hbm.at[sidx0], rows0, sem0).wait()
            pltpu.sync_copy(rows0, acc_sh.at[didx0], add=True)
            fetch(ch + 2, sidx0, didx0, rows0, sem0)
            pltpu.make_async_copy(table_hbm.at[sidx1], rows1, sem1).wait()
            pltpu.sync_copy(rows1, acc_sh.at[didx1], add=True)
            return carry

        lax.fori_loop(0, (EDGE_CHUNKS - 1) // 2, pair_body, 0)
        pltpu.make_async_copy(table_hbm.at[sidx0], rows0, sem0).wait()
        pltpu.sync_copy(rows0, acc_sh.at[didx0], add=True)
        plsc.subcore_barrier()
        for q in range(ROWS_PER_TILE // EDGE_CHUNK):
            rb = zbase + q * EDGE_CHUNK
            pltpu.sync_copy(acc_sh.at[pl.ds(rb, EDGE_CHUNK)], rows0)
            pltpu.sync_copy(rows0, out_hbm.at[cid, pl.ds(rb, EDGE_CHUNK)])

    return edge_agg


_edge_agg_f = _make_edge_agg()

NH = NP // F
_EROWS = 625
_EBLK = 25


def _deg_body(d_ref, out_ref):
    i = pl.program_id(0)
    d = d_ref[...].reshape(-1)
    hi = d // F
    lo = d - hi * F
    a = (hi[:, None] == lax.broadcasted_iota(jnp.int32, (1, NH), 1)
         ).astype(jnp.float32)
    b = (lo[:, None] == lax.broadcasted_iota(jnp.int32, (1, F), 1)
         ).astype(jnp.float32)
    blk = lax.dot_general(a, b, (((0,), (0,)), ((), ())),
                          preferred_element_type=jnp.float32)

    @pl.when(i == 0)
    def _():
        out_ref[...] = jnp.zeros_like(out_ref)

    out_ref[...] += blk


def _deg_mm(dst):
    return pl.pallas_call(
        _deg_body,
        grid=(_EROWS // _EBLK,),
        in_specs=[pl.BlockSpec((1, _EBLK, 512), lambda i: (i, 0, 0))],
        out_specs=pl.BlockSpec((NH, F), lambda i: (0, 0)),
        out_shape=jax.ShapeDtypeStruct((NH, F), jnp.float32),
    )(dst.reshape(_EROWS // _EBLK, _EBLK, 512))


_BN = 1000


def _layer1_body(x_ref, aggp_ref, deg_ref, w_ref, b_ref, h_ref, invd_ref):
    s = aggp_ref[0] + aggp_ref[1]
    invd = 1.0 / jnp.maximum(deg_ref[...], 1.0)
    agg = s * invd
    h = jnp.maximum(
        jnp.dot(x_ref[...] + agg, w_ref[...],
                preferred_element_type=jnp.float32) + b_ref[...], 0.0)
    h_ref[...] = h
    invd_ref[...] = jnp.broadcast_to(invd, (invd.shape[0], 8))


def _layer1(x, aggp, degp, W1, b1):
    return pl.pallas_call(
        _layer1_body,
        grid=(N // _BN,),
        in_specs=[
            pl.BlockSpec((_BN, F), lambda i: (i, 0)),
            pl.BlockSpec((NC, _BN, F), lambda i: (0, i, 0)),
            pl.BlockSpec((_BN, 1), lambda i: (i, 0)),
            pl.BlockSpec((F, F), lambda i: (0, 0)),
            pl.BlockSpec((1, F), lambda i: (0, 0)),
        ],
        out_specs=[
            pl.BlockSpec((_BN, F), lambda i: (i, 0)),
            pl.BlockSpec((_BN, 8), lambda i: (i, 0)),
        ],
        out_shape=[
            jax.ShapeDtypeStruct((N, F), jnp.float32),
            jax.ShapeDtypeStruct((N, 8), jnp.float32),
        ],
    )(x, aggp, degp, W1, b1.reshape(1, F))


def _layer2_body(h_ref, aggp_ref, invd_ref, w_ref, b_ref,
                 wa_ref, wb_ref, wc_ref, bd1_ref, a_ref, b2_ref, c_ref):
    s = aggp_ref[0] + aggp_ref[1]
    agg = s * invd_ref[:, 0:1]
    h2 = jnp.maximum(
        jnp.dot(h_ref[...] + agg, w_ref[...],
                preferred_element_type=jnp.float32) + b_ref[...], 0.0)
    a_ref[...] = jnp.dot(h2, wa_ref[...],
                         preferred_element_type=jnp.float32) + bd1_ref[...]
    b2_ref[...] = jnp.dot(h2, wb_ref[...], preferred_element_type=jnp.float32)
    c_ref[...] = jnp.dot(h2, wc_ref[...], preferred_element_type=jnp.float32)


def _layer2(h, aggp, invd, W2, b2, Wd1, bd1):
    return pl.pallas_call(
        _layer2_body,
        grid=(N // _BN,),
        in_specs=[
            pl.BlockSpec((_BN, F), lambda i: (i, 0)),
            pl.BlockSpec((NC, _BN, F), lambda i: (0, i, 0)),
            pl.BlockSpec((_BN, 8), lambda i: (i, 0)),
            pl.BlockSpec((F, F), lambda i: (0, 0)),
            pl.BlockSpec((1, F), lambda i: (0, 0)),
            pl.BlockSpec((F, D3), lambda i: (0, 0)),
            pl.BlockSpec((F, D3), lambda i: (0, 0)),
            pl.BlockSpec((F, D3), lambda i: (0, 0)),
            pl.BlockSpec((1, D3), lambda i: (0, 0)),
        ],
        out_specs=[
            pl.BlockSpec((_BN, D3), lambda i: (i, 0)),
            pl.BlockSpec((_BN, D3), lambda i: (i, 0)),
            pl.BlockSpec((_BN, D3), lambda i: (i, 0)),
        ],
        out_shape=[
            jax.ShapeDtypeStruct((N, D3), jnp.float32),
            jax.ShapeDtypeStruct((N, D3), jnp.float32),
            jax.ShapeDtypeStruct((N, D3), jnp.float32),
        ],
    )(h, aggp, invd, W2, b2.reshape(1, F),
      Wd1[:F], Wd1[F:2 * F], Wd1[2 * F:], bd1.reshape(1, D3))


@functools.partial(
    pl.kernel,
    mesh=_mesh,
    compiler_params=pltpu.CompilerParams(needs_layout_passes=False),
    out_type=jax.ShapeDtypeStruct((PP, D3), jnp.float32),
    scratch_types=[
        pltpu.VMEM((VCHUNK,), jnp.int32),
        pltpu.VMEM((VCHUNK,), jnp.int32),
        pltpu.VMEM((VCHUNK,), jnp.int32),
        pltpu.VMEM((VCHUNK,), jnp.int32),
        pltpu.VMEM((VCHUNK,), jnp.int32),
        pltpu.VMEM((VCHUNK,), jnp.int32),
        pltpu.VMEM((VCHUNK, D3), jnp.float32),
        pltpu.VMEM((VCHUNK, D3), jnp.float32),
        pltpu.VMEM((VCHUNK, D3), jnp.float32),
        pltpu.VMEM((VCHUNK, D3), jnp.float32),
        pltpu.VMEM((VCHUNK, D3), jnp.float32),
        pltpu.VMEM((VCHUNK, D3), jnp.float32),
        pltpu.SemaphoreType.DMA,
        pltpu.SemaphoreType.DMA,
        pltpu.SemaphoreType.DMA,
        pltpu.SemaphoreType.DMA,
        pltpu.SemaphoreType.DMA,
        pltpu.SemaphoreType.DMA,
    ],
)
def _gather_sum(a_hbm, b_hbm, c_hbm, pi_hbm, si_hbm, di_hbm, out_hbm,
                pi0, si0, di0, pi1, si1, di1,
                a0, b0, c0, a1, b1, c1,
                sa0, sb0, sc0, sa1, sb1, sc1):
    cid = lax.axis_index("c")
    tid = lax.axis_index("s")
    wid = tid * NC + cid
    wbase = wid * VPW
    bufs = ((pi0, si0, di0, a0, b0, c0, sa0, sb0, sc0),
            (pi1, si1, di1, a1, b1, c1, sa1, sb1, sc1))

    def fetch(ch, bfr):
        pi_v, si_v, di_v, a_v, b_v, c_v, sa, sb, sc = bufs[bfr]
        base = pl.multiple_of(wbase + ch * VCHUNK, 8)
        pltpu.sync_copy(pi_hbm.at[pl.ds(base, VCHUNK)], pi_v)
        pltpu.sync_copy(si_hbm.at[pl.ds(base, VCHUNK)], si_v)
        pltpu.sync_copy(di_hbm.at[pl.ds(base, VCHUNK)], di_v)
        pltpu.async_copy(a_hbm.at[pi_v], a_v, sa)
        pltpu.async_copy(b_hbm.at[si_v], b_v, sb)
        pltpu.async_copy(c_hbm.at[di_v], c_v, sc)

    def finish(ch, bfr):
        pi_v, si_v, di_v, a_v, b_v, c_v, sa, sb, sc = bufs[bfr]
        pltpu.make_async_copy(a_hbm.at[pi_v], a_v, sa).wait()
        pltpu.make_async_copy(b_hbm.at[si_v], b_v, sb).wait()
        pltpu.make_async_copy(c_hbm.at[di_v], c_v, sc).wait()

        def sum_row(r, c2):
            for q in range(D3 // L):
                sl = pl.ds(q * L, L)
                a_v[r, sl] = a_v[r, sl] + b_v[r, sl] + c_v[r, sl]
            return c2

        lax.fori_loop(0, VCHUNK, sum_row, 0)
        base = pl.multiple_of(wbase + ch * VCHUNK, 8)
        pltpu.sync_copy(a_v, out_hbm.at[pl.ds(base, VCHUNK)])

    fetch(0, 0)

    def pair_body(j2, carry):
        ch = 2 * j2
        fetch(ch + 1, 1)
        finish(ch, 0)
        fetch(ch + 2, 0)
        finish(ch + 1, 1)
        return carry

    lax.fori_loop(0, (VCHUNKS - 2) // 2, pair_body, 0)
    fetch(VCHUNKS - 1, 1)
    finish(VCHUNKS - 2, 0)
    finish(VCHUNKS - 1, 1)


_VB = 3136


def _decode_body(hs_ref, pi_ref, wd2_ref, bd2_ref, m0_ref, m1_ref):
    i = pl.program_id(0)
    hid = jnp.maximum(hs_ref[...], 0.0)
    logits = jnp.dot(hid, wd2_ref[...],
                     preferred_element_type=jnp.float32) + bd2_ref[...]
    lm = jnp.max(logits, axis=1, keepdims=True)
    lse = lm + jnp.log(jnp.sum(jnp.exp(logits - lm), axis=1, keepdims=True))
    logp = logits - lse
    gidx = i * _VB + lax.broadcasted_iota(jnp.int32, (_VB, 1), 0)
    logp = jnp.where(gidx < P, logp, 0.0)
    pv = pi_ref[...].reshape(-1)
    hi = pv // F
    lo = pv - hi * F
    a = (hi[:, None] == lax.broadcasted_iota(jnp.int32, (1, NH), 1)
         ).astype(jnp.float32)
    b = (lo[:, None] == lax.broadcasted_iota(jnp.int32, (1, F), 1)
         ).astype(jnp.float32)
    m0 = lax.dot_general(a, b * logp[:, 0:1], (((0,), (0,)), ((), ())),
                         preferred_element_type=jnp.float32)
    m1 = lax.dot_general(a, b * logp[:, 1:2], (((0,), (0,)), ((), ())),
                         preferred_element_type=jnp.float32)

    @pl.when(i == 0)
    def _():
        m0_ref[...] = jnp.zeros_like(m0_ref)
        m1_ref[...] = jnp.zeros_like(m1_ref)

    m0_ref[...] += m0
    m1_ref[...] += m1


def _decode(hidsum, pi, Wd2, bd2):
    return pl.pallas_call(
        _decode_body,
        grid=(PP // _VB,),
        in_specs=[
            pl.BlockSpec((_VB, D3), lambda i: (i, 0)),
            pl.BlockSpec((1, 1, _VB), lambda i: (i, 0, 0)),
            pl.BlockSpec((D3, OUT), lambda i: (0, 0)),
            pl.BlockSpec((1, OUT), lambda i: (0, 0)),
        ],
        out_specs=[
            pl.BlockSpec((NH, F), lambda i: (0, 0)),
            pl.BlockSpec((NH, F), lambda i: (0, 0)),
        ],
        out_shape=[
            jax.ShapeDtypeStruct((NH, F), jnp.float32),
            jax.ShapeDtypeStruct((NH, F), jnp.float32),
        ],
    )(hidsum, pi.reshape(PP // _VB, 1, _VB), Wd2, bd2.reshape(1, OUT))


def _first(x):
    return x[0] if isinstance(x, (tuple, list)) else x


def kernel(x, edge_index, original, y, nodes, variants,
           W1, b1, W2, b2, Wd1, bd1, Wd2, bd2):
    del original, y, nodes
    src = edge_index[0]
    dst = edge_index[1]

    aggp1 = _first(_edge_agg_f(x, src, dst))
    deg = _deg_mm(dst).reshape(NP, 1)
    h, invd = _layer1(x, aggp1, deg, W1, b1)
    aggp2 = _first(_edge_agg_f(h, src, dst))
    A, B, C = _layer2(h, aggp2, invd, W2, b2, Wd1, bd1)

    pad = jnp.zeros((PP - P,), jnp.int32)
    pi = jnp.concatenate([variants[0], pad])
    si = jnp.concatenate([variants[1], pad])
    di = jnp.concatenate([variants[2], pad])
    hidsum = _first(_gather_sum(A, B, C, pi, si, di))
    m0, m1 = _decode(hidsum, pi, Wd2, bd2)
    return jnp.stack([m0.reshape(NP)[:N], m1.reshape(NP)[:N]], axis=1)


# speedup vs baseline: 5.2147x; 1.0008x over previous
import functools

import jax
import jax.numpy as jnp
from jax import lax
from jax.experimental import pallas as pl
from jax.experimental.pallas import tpu as pltpu
from jax.experimental.pallas import tpu_sc as plsc

N = 10000
E = 320000
P = 100000
F = 128
OUT = 2
NC = 2
NS = 16
NW = NC * NS
L = 16

NP = 10240
ROWS_PER_TILE = NP // NS
EDGE_CHUNK = 80
EDGES_PER_W = E // NW
EDGE_CHUNKS = EDGES_PER_W // EDGE_CHUNK

VPW = 3136
PP = VPW * NW
VCHUNK = 32
VCHUNKS = VPW // VCHUNK
D3 = 3 * F

_mesh = plsc.VectorSubcoreMesh(core_axis_name="c", subcore_axis_name="s",
                               num_cores=NC, num_subcores=NS)


def _make_edge_agg():
    @functools.partial(
        pl.kernel,
        mesh=_mesh,
        compiler_params=pltpu.CompilerParams(needs_layout_passes=False),
        out_type=jax.ShapeDtypeStruct((NC, NP, F), jnp.float32),
        scratch_types=[
            pltpu.VMEM((EDGE_CHUNK,), jnp.int32),
            pltpu.VMEM((EDGE_CHUNK,), jnp.int32),
            pltpu.VMEM((EDGE_CHUNK,), jnp.int32),
            pltpu.VMEM((EDGE_CHUNK,), jnp.int32),
            pltpu.VMEM((EDGE_CHUNK, F), jnp.float32),
            pltpu.VMEM((EDGE_CHUNK, F), jnp.float32),
            pltpu.VMEM_SHARED((NP, F), jnp.float32),
            pltpu.SemaphoreType.DMA,
            pltpu.SemaphoreType.DMA,
        ],
    )
    def edge_agg(table_hbm, src_hbm, dst_hbm, out_hbm,
                 sidx0, didx0, sidx1, didx1, rows0, rows1, acc_sh,
                 sem0, sem1):
        cid = lax.axis_index("c")
        tid = lax.axis_index("s")
        wid = tid * NC + cid
        wbase = wid * EDGES_PER_W
        zbase = pl.multiple_of(tid * ROWS_PER_TILE, 8)
        zv = jnp.zeros((L,), jnp.float32)

        def zrow(r, carry):
            for q in range(F // L):
                rows0[r, pl.ds(q * L, L)] = zv
            return carry

        lax.fori_loop(0, EDGE_CHUNK, zrow, 0)
        for q in range(ROWS_PER_TILE // EDGE_CHUNK):
            pltpu.sync_copy(
                rows0, acc_sh.at[pl.ds(zbase + q * EDGE_CHUNK, EDGE_CHUNK)])
        plsc.subcore_barrier()

        def fetch(ch, sidx, didx, rows, sem):
            base = pl.multiple_of(wbase + ch * EDGE_CHUNK, 8)
            pltpu.sync_copy(src_hbm.at[pl.ds(base, EDGE_CHUNK)], sidx)
            pltpu.sync_copy(dst_hbm.at[pl.ds(base, EDGE_CHUNK)], didx)
            return pltpu.async_copy(table_hbm.at[sidx], rows, sem)

        fetch(0, sidx0, didx0, rows0, sem0)

        def pair_body(j2, carry):
            ch = 2 * j2
            fetch(ch + 1, sidx1, didx1, rows1, sem1)
            pltpu.make_async_copy(table_hbm.at[sidx0], rows0, sem0).wait()
            pltpu.sync_copy(rows0, acc_sh.at[didx0], add=True)
            fetch(ch + 2, sidx0, didx0, rows0, sem0)
            pltpu.make_async_copy(table_hbm.at[sidx1], rows1, sem1).wait()
            pltpu.sync_copy(rows1, acc_sh.at[didx1], add=True)
            return carry

        lax.fori_loop(0, (EDGE_CHUNKS - 1) // 2, pair_body, 0)
        pltpu.make_async_copy(table_hbm.at[sidx0], rows0, sem0).wait()
        pltpu.sync_copy(rows0, acc_sh.at[didx0], add=True)
        plsc.subcore_barrier()
        for q in range(ROWS_PER_TILE // EDGE_CHUNK):
            rb = zbase + q * EDGE_CHUNK
            pltpu.sync_copy(acc_sh.at[pl.ds(rb, EDGE_CHUNK)], rows0)
            pltpu.sync_copy(rows0, out_hbm.at[cid, pl.ds(rb, EDGE_CHUNK)])

    return edge_agg


_edge_agg_f = _make_edge_agg()

NH = NP // F
_EROWS = 625
_EBLK = 25


def _deg_body(d_ref, out_ref):
    i = pl.program_id(0)
    d = d_ref[...].reshape(-1)
    hi = d // F
    lo = d - hi * F
    a = (hi[:, None] == lax.broadcasted_iota(jnp.int32, (1, NH), 1)
         ).astype(jnp.float32)
    b = (lo[:, None] == lax.broadcasted_iota(jnp.int32, (1, F), 1)
         ).astype(jnp.float32)
    blk = lax.dot_general(a, b, (((0,), (0,)), ((), ())),
                          preferred_element_type=jnp.float32)

    @pl.when(i == 0)
    def _():
        out_ref[...] = jnp.zeros_like(out_ref)

    out_ref[...] += blk


def _deg_mm(dst):
    return pl.pallas_call(
        _deg_body,
        grid=(_EROWS // _EBLK,),
        in_specs=[pl.BlockSpec((1, _EBLK, 512), lambda i: (i, 0, 0))],
        out_specs=pl.BlockSpec((NH, F), lambda i: (0, 0)),
        out_shape=jax.ShapeDtypeStruct((NH, F), jnp.float32),
    )(dst.reshape(_EROWS // _EBLK, _EBLK, 512))


_BN = 1000


def _layer1_body(x_ref, aggp_ref, deg_ref, w_ref, b_ref, h_ref, invd_ref):
    s = aggp_ref[0] + aggp_ref[1]
    invd = 1.0 / jnp.maximum(deg_ref[...], 1.0)
    agg = s * invd
    h = jnp.maximum(
        jnp.dot(x_ref[...] + agg, w_ref[...],
                preferred_element_type=jnp.float32) + b_ref[...], 0.0)
    h_ref[...] = h
    invd_ref[...] = jnp.broadcast_to(invd, (invd.shape[0], 8))


def _layer1(x, aggp, degp, W1, b1):
    return pl.pallas_call(
        _layer1_body,
        grid=(N // _BN,),
        in_specs=[
            pl.BlockSpec((_BN, F), lambda i: (i, 0)),
            pl.BlockSpec((NC, _BN, F), lambda i: (0, i, 0)),
            pl.BlockSpec((_BN, 1), lambda i: (i, 0)),
            pl.BlockSpec((F, F), lambda i: (0, 0)),
            pl.BlockSpec((1, F), lambda i: (0, 0)),
        ],
        out_specs=[
            pl.BlockSpec((_BN, F), lambda i: (i, 0)),
            pl.BlockSpec((_BN, 8), lambda i: (i, 0)),
        ],
        out_shape=[
            jax.ShapeDtypeStruct((N, F), jnp.float32),
            jax.ShapeDtypeStruct((N, 8), jnp.float32),
        ],
    )(x, aggp, degp, W1, b1.reshape(1, F))


def _layer2_body(h_ref, aggp_ref, invd_ref, w_ref, b_ref,
                 wa_ref, wb_ref, wc_ref, bd1_ref, a_ref, b2_ref, c_ref):
    s = aggp_ref[0] + aggp_ref[1]
    agg = s * invd_ref[:, 0:1]
    h2 = jnp.maximum(
        jnp.dot(h_ref[...] + agg, w_ref[...],
                preferred_element_type=jnp.float32) + b_ref[...], 0.0)
    a_ref[...] = jnp.dot(h2, wa_ref[...],
                         preferred_element_type=jnp.float32) + bd1_ref[...]
    b2_ref[...] = jnp.dot(h2, wb_ref[...], preferred_element_type=jnp.float32)
    c_ref[...] = jnp.dot(h2, wc_ref[...], preferred_element_type=jnp.float32)


def _layer2(h, aggp, invd, W2, b2, Wd1, bd1):
    return pl.pallas_call(
        _layer2_body,
        grid=(N // _BN,),
        in_specs=[
            pl.BlockSpec((_BN, F), lambda i: (i, 0)),
            pl.BlockSpec((NC, _BN, F), lambda i: (0, i, 0)),
            pl.BlockSpec((_BN, 8), lambda i: (i, 0)),
            pl.BlockSpec((F, F), lambda i: (0, 0)),
            pl.BlockSpec((1, F), lambda i: (0, 0)),
            pl.BlockSpec((F, D3), lambda i: (0, 0)),
            pl.BlockSpec((F, D3), lambda i: (0, 0)),
            pl.BlockSpec((F, D3), lambda i: (0, 0)),
            pl.BlockSpec((1, D3), lambda i: (0, 0)),
        ],
        out_specs=[
            pl.BlockSpec((_BN, D3), lambda i: (i, 0)),
            pl.BlockSpec((_BN, D3), lambda i: (i, 0)),
            pl.BlockSpec((_BN, D3), lambda i: (i, 0)),
        ],
        out_shape=[
            jax.ShapeDtypeStruct((N, D3), jnp.float32),
            jax.ShapeDtypeStruct((N, D3), jnp.float32),
            jax.ShapeDtypeStruct((N, D3), jnp.float32),
        ],
    )(h, aggp, invd, W2, b2.reshape(1, F),
      Wd1[:F], Wd1[F:2 * F], Wd1[2 * F:], bd1.reshape(1, D3))


@functools.partial(
    pl.kernel,
    mesh=_mesh,
    compiler_params=pltpu.CompilerParams(needs_layout_passes=False),
    out_type=jax.ShapeDtypeStruct((PP, D3), jnp.float32),
    scratch_types=[
        pltpu.VMEM((VCHUNK,), jnp.int32),
        pltpu.VMEM((VCHUNK,), jnp.int32),
        pltpu.VMEM((VCHUNK,), jnp.int32),
        pltpu.VMEM((VCHUNK,), jnp.int32),
        pltpu.VMEM((VCHUNK,), jnp.int32),
        pltpu.VMEM((VCHUNK,), jnp.int32),
        pltpu.VMEM((VCHUNK, D3), jnp.float32),
        pltpu.VMEM((VCHUNK, D3), jnp.float32),
        pltpu.VMEM((VCHUNK, D3), jnp.float32),
        pltpu.VMEM((VCHUNK, D3), jnp.float32),
        pltpu.VMEM((VCHUNK, D3), jnp.float32),
        pltpu.VMEM((VCHUNK, D3), jnp.float32),
        pltpu.SemaphoreType.DMA,
        pltpu.SemaphoreType.DMA,
        pltpu.SemaphoreType.DMA,
        pltpu.SemaphoreType.DMA,
        pltpu.SemaphoreType.DMA,
        pltpu.SemaphoreType.DMA,
    ],
)
def _gather_sum(a_hbm, b_hbm, c_hbm, pi_hbm, si_hbm, di_hbm, out_hbm,
                pi0, si0, di0, pi1, si1, di1,
                a0, b0, c0, a1, b1, c1,
                sa0, sb0, sc0, sa1, sb1, sc1):
    cid = lax.axis_index("c")
    tid = lax.axis_index("s")
    wid = tid * NC + cid
    wbase = wid * VPW
    bufs = ((pi0, si0, di0, a0, b0, c0, sa0, sb0, sc0),
            (pi1, si1, di1, a1, b1, c1, sa1, sb1, sc1))

    def fetch(ch, bfr):
        pi_v, si_v, di_v, a_v, b_v, c_v, sa, sb, sc = bufs[bfr]
        base = pl.multiple_of(wbase + ch * VCHUNK, 8)
        pltpu.sync_copy(pi_hbm.at[pl.ds(base, VCHUNK)], pi_v)
        pltpu.sync_copy(si_hbm.at[pl.ds(base, VCHUNK)], si_v)
        pltpu.sync_copy(di_hbm.at[pl.ds(base, VCHUNK)], di_v)
        pltpu.async_copy(a_hbm.at[pi_v], a_v, sa)
        pltpu.async_copy(b_hbm.at[si_v], b_v, sb)
        pltpu.async_copy(c_hbm.at[di_v], c_v, sc)

    def finish(ch, bfr):
        pi_v, si_v, di_v, a_v, b_v, c_v, sa, sb, sc = bufs[bfr]
        pltpu.make_async_copy(a_hbm.at[pi_v], a_v, sa).wait()
        pltpu.make_async_copy(b_hbm.at[si_v], b_v, sb).wait()
        pltpu.make_async_copy(c_hbm.at[di_v], c_v, sc).wait()

        def sum_row(r, c2):
            for q in range(D3 // L):
                sl = pl.ds(q * L, L)
                a_v[r, sl] = a_v[r, sl] + b_v[r, sl] + c_v[r, sl]
            return c2

        lax.fori_loop(0, VCHUNK, sum_row, 0)
        base = pl.multiple_of(wbase + ch * VCHUNK, 8)
        pltpu.sync_copy(a_v, out_hbm.at[pl.ds(base, VCHUNK)])

    fetch(0, 0)

    def pair_body(j2, carry):
        ch = 2 * j2
        fetch(ch + 1, 1)
        finish(ch, 0)
        fetch(ch + 2, 0)
        finish(ch + 1, 1)
        return carry

    lax.fori_loop(0, (VCHUNKS - 2) // 2, pair_body, 0)
    fetch(VCHUNKS - 1, 1)
    finish(VCHUNKS - 2, 0)
    finish(VCHUNKS - 1, 1)


_VB = 3136


def _decode_body(hs_ref, pi_ref, wd2_ref, bd2_ref, m0_ref, m1_ref):
    i = pl.program_id(0)
    hid = jnp.maximum(hs_ref[...], 0.0)
    logits = jnp.dot(hid, wd2_ref[...],
                     preferred_element_type=jnp.float32) + bd2_ref[...]
    lm = jnp.max(logits, axis=1, keepdims=True)
    lse = lm + jnp.log(jnp.sum(jnp.exp(logits - lm), axis=1, keepdims=True))
    logp = logits - lse
    gidx = i * _VB + lax.broadcasted_iota(jnp.int32, (_VB, 1), 0)
    logp = jnp.where(gidx < P, logp, 0.0)
    pv = pi_ref[...].reshape(-1)
    hi = pv // F
    lo = pv - hi * F
    a = (hi[:, None] == lax.broadcasted_iota(jnp.int32, (1, NH), 1)
         ).astype(jnp.float32)
    b = (lo[:, None] == lax.broadcasted_iota(jnp.int32, (1, F), 1)
         ).astype(jnp.float32)
    m0 = lax.dot_general(a, b * logp[:, 0:1], (((0,), (0,)), ((), ())),
                         preferred_element_type=jnp.float32)
    m1 = lax.dot_general(a, b * logp[:, 1:2], (((0,), (0,)), ((), ())),
                         preferred_element_type=jnp.float32)

    @pl.when(i == 0)
    def _():
        m0_ref[...] = jnp.zeros_like(m0_ref)
        m1_ref[...] = jnp.zeros_like(m1_ref)

    m0_ref[...] += m0
    m1_ref[...] += m1


def _decode(hidsum, pi, Wd2, bd2):
    return pl.pallas_call(
        _decode_body,
        grid=(PP // _VB,),
        in_specs=[
            pl.BlockSpec((_VB, D3), lambda i: (i, 0)),
            pl.BlockSpec((1, 1, _VB), lambda i: (i, 0, 0)),
            pl.BlockSpec((D3, OUT), lambda i: (0, 0)),
            pl.BlockSpec((1, OUT), lambda i: (0, 0)),
        ],
        out_specs=[
            pl.BlockSpec((NH, F), lambda i: (0, 0)),
            pl.BlockSpec((NH, F), lambda i: (0, 0)),
        ],
        out_shape=[
            jax.ShapeDtypeStruct((NH, F), jnp.float32),
            jax.ShapeDtypeStruct((NH, F), jnp.float32),
        ],
    )(hidsum, pi.reshape(PP // _VB, 1, _VB), Wd2, bd2.reshape(1, OUT))


def _first(x):
    return x[0] if isinstance(x, (tuple, list)) else x


def kernel(x, edge_index, original, y, nodes, variants,
           W1, b1, W2, b2, Wd1, bd1, Wd2, bd2):
    del original, y, nodes
    src = edge_index[0]
    dst = edge_index[1]

    aggp1 = _first(_edge_agg_f(x, src, dst))
    deg = _deg_mm(dst).reshape(NP, 1)
    h, invd = _layer1(x, aggp1, deg, W1, b1)
    aggp2 = _first(_edge_agg_f(h, src, dst))
    A, B, C = _layer2(h, aggp2, invd, W2, b2, Wd1, bd1)

    pad = jnp.zeros((PP - P,), jnp.int32)
    pi = jnp.concatenate([variants[0], pad])
    si = jnp.concatenate([variants[1], pad])
    di = jnp.concatenate([variants[2], pad])
    hidsum = _first(_gather_sum(A, B, C, pi, si, di))
    m0, m1 = _decode(hidsum, pi, Wd2, bd2)
    return jnp.stack([m0.reshape(NP)[:N], m1.reshape(NP)[:N]], axis=1)
